# baseline (speedup 1.0000x reference)
import functools

import jax
import jax.numpy as jnp
from jax import lax
from jax.experimental import pallas as pl
from jax.experimental.pallas import tpu as pltpu
from jax.experimental.pallas import tpu_sc as plsc

_N_NODES = 10000
_ROWS = 10000
_D = 128
_CL = 16
_E = 320000
_TILES = 16
_W = 128
_NWIN = _E // _W
_NSTEP = 39
_RPT = _ROWS // _TILES


def _accumulate(idx2d, ea_hbm, acc_f, acc_c,
                row_a, row_b, cba, cbb, ones_v,
                gsa, gsb, ssa, ssb, ica, icb, tile):
    w0 = tile * 156 + jnp.minimum(tile, 4)
    has_tail = tile < 4

    def issue_rows(j, rowv, sem):
        e0 = pl.multiple_of((w0 + j) * _W, 8)
        pltpu.async_copy(ea_hbm.at[pl.ds(e0, _W)], rowv, sem)

    def wait_rows(rowv, sem):
        pltpu.make_async_copy(ea_hbm.at[pl.ds(0, _W)], rowv, sem).wait()

    def issue_chunk(j, cb, sem):
        pltpu.async_copy(idx2d.at[pl.ds(w0 + j, 2)], cb, sem)

    def wait_chunk(cb, sem):
        pltpu.make_async_copy(idx2d.at[pl.ds(0, 2)], cb, sem).wait()

    def issue_scatter(idxv, rowv, sem):
        pltpu.async_copy(rowv, acc_f.at[idxv], sem, add=True)
        pltpu.async_copy(ones_v, acc_c.at[idxv], sem, add=True)

    def wait_scatter(idxv, rowv, sem):
        pltpu.make_async_copy(rowv, acc_f.at[idxv], sem).wait()
        pltpu.make_async_copy(ones_v, acc_c.at[idxv], sem).wait()

    issue_chunk(0, cba, ica)
    issue_rows(0, row_a, gsa)
    issue_rows(1, row_b, gsb)

    def body(m, _):
        j0 = 4 * m
        wait_chunk(cba, ica)
        wait_rows(row_a, gsa)
        issue_scatter(cba.at[0], row_a, ssa)

        @pl.when(m > 0)
        def _():
            wait_scatter(cbb.at[1], row_b, ssb)
            issue_rows(j0 + 1, row_b, gsb)

        issue_chunk(j0 + 2, cbb, icb)

        wait_rows(row_b, gsb)
        issue_scatter(cba.at[1], row_b, ssb)
        wait_scatter(cba.at[0], row_a, ssa)
        issue_rows(j0 + 2, row_a, gsa)
        wait_chunk(cbb, icb)
        wait_rows(row_a, gsa)
        issue_scatter(cbb.at[0], row_a, ssa)
        wait_scatter(cba.at[1], row_b, ssb)

        @pl.when((m + 1 < _NSTEP) | has_tail)
        def _():
            issue_chunk(j0 + 4, cba, ica)

        issue_rows(j0 + 3, row_b, gsb)
        wait_rows(row_b, gsb)
        issue_scatter(cbb.at[1], row_b, ssb)
        wait_scatter(cbb.at[0], row_a, ssa)

        @pl.when((4 * m + 4 < _NSTEP * 4) | has_tail)
        def _():
            issue_rows(j0 + 4, row_a, gsa)

        return 0

    lax.fori_loop(0, _NSTEP, body, 0)
    wait_scatter(cbb.at[1], row_b, ssb)

    @pl.when(has_tail)
    def _():
        wait_chunk(cba, ica)
        wait_rows(row_a, gsa)
        pltpu.sync_copy(row_a, acc_f.at[cba.at[0]], add=True)
        pltpu.sync_copy(ones_v, acc_c.at[cba.at[0]], add=True)


def _sc_pool_body(ea1, idx2d1, ea2, idx2d2, zf, zc, ones2d,
                  s1, c1, s2, c2,
                  acc_f, acc_c, row_a, row_b, cba, cbb,
                  ones_v, gsa, gsb, ssa, ssb, ica, icb):
    c = lax.axis_index("c")
    s = lax.axis_index("s")
    rows = pl.ds(s * _RPT, _RPT)
    pltpu.sync_copy(zf.at[rows], acc_f.at[rows])
    pltpu.sync_copy(zc.at[rows], acc_c.at[rows])
    pltpu.sync_copy(ones2d, ones_v)
    plsc.subcore_barrier()

    @pl.when(c == 0)
    def _():
        _accumulate(idx2d1, ea1, acc_f, acc_c, row_a, row_b, cba, cbb,
                    ones_v, gsa, gsb, ssa, ssb, ica, icb, s)

    @pl.when(c == 1)
    def _():
        _accumulate(idx2d2, ea2, acc_f, acc_c, row_a, row_b, cba, cbb,
                    ones_v, gsa, gsb, ssa, ssb, ica, icb, s)

    plsc.subcore_barrier()

    @pl.when(c == 0)
    def _():
        pltpu.sync_copy(acc_f.at[rows], s1.at[rows])
        pltpu.sync_copy(acc_c.at[rows], c1.at[rows])

    @pl.when(c == 1)
    def _():
        pltpu.sync_copy(acc_f.at[rows], s2.at[rows])
        pltpu.sync_copy(acc_c.at[rows], c2.at[rows])


_F32 = jnp.float32

_sc_pool = functools.partial(
    pl.kernel,
    out_type=(
        jax.ShapeDtypeStruct((_ROWS, _D), _F32),
        jax.ShapeDtypeStruct((_ROWS, _CL), _F32),
        jax.ShapeDtypeStruct((_ROWS, _D), _F32),
        jax.ShapeDtypeStruct((_ROWS, _CL), _F32),
    ),
    mesh=plsc.VectorSubcoreMesh(core_axis_name="c", subcore_axis_name="s"),
    compiler_params=pltpu.CompilerParams(use_tc_tiling_on_sc=False),
    scratch_types=(
        pltpu.VMEM_SHARED((_ROWS, _D), _F32),
        pltpu.VMEM_SHARED((_ROWS, _CL), _F32),
        pltpu.VMEM((_W, _D), _F32),
        pltpu.VMEM((_W, _D), _F32),
        pltpu.VMEM((2, _W), jnp.int32),
        pltpu.VMEM((2, _W), jnp.int32),
        pltpu.VMEM((_W, _CL), _F32),
        pltpu.SemaphoreType.DMA,
        pltpu.SemaphoreType.DMA,
        pltpu.SemaphoreType.DMA,
        pltpu.SemaphoreType.DMA,
        pltpu.SemaphoreType.DMA,
        pltpu.SemaphoreType.DMA,
    ),
)(_sc_pool_body)


_BR = 1000


def _combine_body(s1, c1, s2, c2, o):
    n1 = jnp.maximum(c1[:, 0:1], 1.0)
    n2 = jnp.maximum(c2[:, 0:1], 1.0)
    o[...] = s1[...] / n1 + s2[...] / n2


def _combine(s1, c1, s2, c2):
    return pl.pallas_call(
        _combine_body,
        grid=(_N_NODES // _BR,),
        in_specs=[
            pl.BlockSpec((_BR, _D), lambda i: (i, 0)),
            pl.BlockSpec((_BR, _CL), lambda i: (i, 0)),
            pl.BlockSpec((_BR, _D), lambda i: (i, 0)),
            pl.BlockSpec((_BR, _CL), lambda i: (i, 0)),
        ],
        out_specs=pl.BlockSpec((_BR, _D), lambda i: (i, 0)),
        out_shape=jax.ShapeDtypeStruct((_N_NODES, _D), _F32),
    )(s1, c1, s2, c2)


def kernel(edge_attr, edge_attr2, edge_index, edge_index2, num_nodes):
    del num_nodes
    idx2d1 = edge_index[0].astype(jnp.int32).reshape(_NWIN, _W)
    idx2d2 = edge_index2[0].astype(jnp.int32).reshape(_NWIN, _W)
    zf = jnp.zeros((_ROWS, _D), _F32)
    zc = jnp.zeros((_ROWS, _CL), _F32)
    ones2d = jnp.ones((_W, _CL), _F32)
    s1, c1, s2, c2 = _sc_pool(edge_attr, idx2d1, edge_attr2, idx2d2,
                              zf, zc, ones2d)
    return _combine(s1, c1, s2, c2)


# speedup vs baseline: 1.0017x; 1.0017x over previous
import functools

import jax
import jax.numpy as jnp
from jax import lax
from jax.experimental import pallas as pl
from jax.experimental.pallas import tpu as pltpu
from jax.experimental.pallas import tpu_sc as plsc

_N_NODES = 10000
_ROWS = 10000
_D = 128
_CL = 16
_E = 320000
_TILES = 16
_EPT = _E // _TILES
_W = 128
_NFULL = _EPT // _W
_REM = _EPT - _NFULL * _W
_RPT = _ROWS // _TILES


def _accumulate(idx_hbm, ea_hbm, acc_f, acc_c,
                idx_a, row_a, idx_b, row_b, ones_v, idx_r, row_r, ones_r,
                gsa, gsb, ssa, ssb, tile):
    base = tile * _EPT

    def issue_gather(w, idxv, rowv, sem):
        e0 = pl.multiple_of(base + w * _W, 8)
        pltpu.async_copy(idx_hbm.at[pl.ds(e0, _W)], idxv, sem)
        pltpu.async_copy(ea_hbm.at[pl.ds(e0, _W)], rowv, sem)

    def wait_gather(idxv, rowv, sem):
        pltpu.make_async_copy(idx_hbm.at[pl.ds(0, _W)], idxv, sem).wait()
        pltpu.make_async_copy(ea_hbm.at[pl.ds(0, _W)], rowv, sem).wait()

    def issue_scatter(idxv, rowv, sem):
        pltpu.async_copy(rowv, acc_f.at[idxv], sem, add=True)
        pltpu.async_copy(ones_v, acc_c.at[idxv], sem, add=True)

    def wait_scatter(idxv, rowv, sem):
        pltpu.make_async_copy(rowv, acc_f.at[idxv], sem).wait()
        pltpu.make_async_copy(ones_v, acc_c.at[idxv], sem).wait()

    issue_gather(0, idx_a, row_a, gsa)
    issue_gather(1, idx_b, row_b, gsb)

    def body(i, _):
        w = 2 * i
        wait_gather(idx_a, row_a, gsa)
        issue_scatter(idx_a, row_a, ssa)

        @pl.when(i > 0)
        def _():
            wait_scatter(idx_b, row_b, ssb)
            issue_gather(w + 1, idx_b, row_b, gsb)

        wait_gather(idx_b, row_b, gsb)
        issue_scatter(idx_b, row_b, ssb)
        wait_scatter(idx_a, row_a, ssa)

        @pl.when(w + 2 < _NFULL)
        def _():
            issue_gather(w + 2, idx_a, row_a, gsa)

        return 0

    lax.fori_loop(0, _NFULL // 2, body, 0)
    e0 = base + _NFULL * _W
    pltpu.async_copy(idx_hbm.at[pl.ds(e0, _REM)], idx_r, gsa)
    pltpu.async_copy(ea_hbm.at[pl.ds(e0, _REM)], row_r, gsa)
    wait_scatter(idx_b, row_b, ssb)
    pltpu.make_async_copy(idx_hbm.at[pl.ds(0, _REM)], idx_r, gsa).wait()
    pltpu.make_async_copy(ea_hbm.at[pl.ds(0, _REM)], row_r, gsa).wait()
    pltpu.sync_copy(row_r, acc_f.at[idx_r], add=True)
    pltpu.sync_copy(ones_r, acc_c.at[idx_r], add=True)


def _sc_pool_body(ea1, idx1, ea2, idx2, zf, zc, ones2d,
                  s1, c1, s2, c2,
                  acc_f, acc_c, idx_a, row_a, idx_b, row_b,
                  ones_v, idx_r, row_r, ones_r, gsa, gsb, ssa, ssb):
    c = lax.axis_index("c")
    s = lax.axis_index("s")
    rows = pl.ds(s * _RPT, _RPT)
    pltpu.sync_copy(zf.at[rows], acc_f.at[rows])
    pltpu.sync_copy(zc.at[rows], acc_c.at[rows])
    pltpu.sync_copy(ones2d, ones_v)
    pltpu.sync_copy(ones2d.at[pl.ds(0, _REM)], ones_r)
    plsc.subcore_barrier()

    @pl.when(c == 0)
    def _():
        _accumulate(idx1, ea1, acc_f, acc_c,
                    idx_a, row_a, idx_b, row_b, ones_v, idx_r, row_r, ones_r,
                    gsa, gsb, ssa, ssb, s)

    @pl.when(c == 1)
    def _():
        _accumulate(idx2, ea2, acc_f, acc_c,
                    idx_a, row_a, idx_b, row_b, ones_v, idx_r, row_r, ones_r,
                    gsa, gsb, ssa, ssb, s)

    plsc.subcore_barrier()

    @pl.when(c == 0)
    def _():
        pltpu.sync_copy(acc_f.at[rows], s1.at[rows])
        pltpu.sync_copy(acc_c.at[rows], c1.at[rows])

    @pl.when(c == 1)
    def _():
        pltpu.sync_copy(acc_f.at[rows], s2.at[rows])
        pltpu.sync_copy(acc_c.at[rows], c2.at[rows])


_F32 = jnp.float32

_sc_pool = functools.partial(
    pl.kernel,
    out_type=(
        jax.ShapeDtypeStruct((_ROWS, _D), _F32),
        jax.ShapeDtypeStruct((_ROWS, _CL), _F32),
        jax.ShapeDtypeStruct((_ROWS, _D), _F32),
        jax.ShapeDtypeStruct((_ROWS, _CL), _F32),
    ),
    mesh=plsc.VectorSubcoreMesh(core_axis_name="c", subcore_axis_name="s"),
    compiler_params=pltpu.CompilerParams(use_tc_tiling_on_sc=False),
    scratch_types=(
        pltpu.VMEM_SHARED((_ROWS, _D), _F32),
        pltpu.VMEM_SHARED((_ROWS, _CL), _F32),
        pltpu.VMEM((_W,), jnp.int32),
        pltpu.VMEM((_W, _D), _F32),
        pltpu.VMEM((_W,), jnp.int32),
        pltpu.VMEM((_W, _D), _F32),
        pltpu.VMEM((_W, _CL), _F32),
        pltpu.VMEM((_REM,), jnp.int32),
        pltpu.VMEM((_REM, _D), _F32),
        pltpu.VMEM((_REM, _CL), _F32),
        pltpu.SemaphoreType.DMA,
        pltpu.SemaphoreType.DMA,
        pltpu.SemaphoreType.DMA,
        pltpu.SemaphoreType.DMA,
    ),
)(_sc_pool_body)


_BR = 1000


def _combine_body(s1, c1, s2, c2, o):
    n1 = jnp.maximum(c1[:, 0:1], 1.0)
    n2 = jnp.maximum(c2[:, 0:1], 1.0)
    o[...] = s1[...] / n1 + s2[...] / n2


def _combine(s1, c1, s2, c2):
    return pl.pallas_call(
        _combine_body,
        grid=(_N_NODES // _BR,),
        in_specs=[
            pl.BlockSpec((_BR, _D), lambda i: (i, 0)),
            pl.BlockSpec((_BR, _CL), lambda i: (i, 0)),
            pl.BlockSpec((_BR, _D), lambda i: (i, 0)),
            pl.BlockSpec((_BR, _CL), lambda i: (i, 0)),
        ],
        out_specs=pl.BlockSpec((_BR, _D), lambda i: (i, 0)),
        out_shape=jax.ShapeDtypeStruct((_N_NODES, _D), _F32),
    )(s1, c1, s2, c2)


def kernel(edge_attr, edge_attr2, edge_index, edge_index2, num_nodes):
    del num_nodes
    idx1 = edge_index[0].astype(jnp.int32)
    idx2 = edge_index2[0].astype(jnp.int32)
    zf = jnp.zeros((_ROWS, _D), _F32)
    zc = jnp.zeros((_ROWS, _CL), _F32)
    ones2d = jnp.ones((_W, _CL), _F32)
    s1, c1, s2, c2 = _sc_pool(edge_attr, idx1, edge_attr2, idx2, zf, zc, ones2d)
    return _combine(s1, c1, s2, c2)
